# baseline (speedup 1.0000x reference)
import jax
import jax.numpy as jnp
from jax.experimental import pallas as pl

N = 10000
H = 8
DH = 16


def _ln(x, w):
    mu = jnp.mean(x, axis=-1, keepdims=True)
    var = jnp.mean((x - mu) ** 2, axis=-1, keepdims=True)
    return (x - mu) / jnp.sqrt(var + 1e-5) * w


def _id_body(x_ref, o_ref):
    o_ref[...] = x_ref[...]


def kernel(x, edges_idx, ln1_w, Wl, bl, Wr, br, att, attn_bias, ln2_w, W_fc, W_proj):
    h = _ln(x, ln1_w)
    src, dst = edges_idx[0], edges_idx[1]
    xl = (h @ Wl + bl).reshape(N, H, DH)
    xr = (h @ Wr + br).reshape(N, H, DH)
    xj = xl[src]
    xi = xr[dst]
    e = jax.nn.leaky_relu(xi + xj, negative_slope=0.2)
    score = jnp.sum(e * att[None, :, :], axis=-1)
    m = jax.ops.segment_max(score, dst, num_segments=N)
    m = jnp.where(jnp.isfinite(m), m, 0.0)
    p = jnp.exp(score - m[dst])
    denom = jax.ops.segment_sum(p, dst, num_segments=N)
    alpha = p / (denom[dst] + 1e-16)
    out = jax.ops.segment_sum(alpha[:, :, None] * xj, dst, num_segments=N)
    x = x + out.reshape(N, H * DH) + attn_bias
    hh = _ln(x, ln2_w)
    hh = jax.nn.gelu(hh @ W_fc, approximate=False) @ W_proj
    y = x + hh
    return pl.pallas_call(
        _id_body, out_shape=jax.ShapeDtypeStruct(y.shape, y.dtype)
    )(y)


# speedup vs baseline: 31.9103x; 31.9103x over previous
import functools

import jax
import jax.numpy as jnp
from jax import lax
from jax.experimental import pallas as pl
from jax.experimental.pallas import tpu as pltpu
from jax.experimental.pallas import tpu_sc as plsc

NN = 10000
EE = 320000
DD = 128
HH = 8
DHD = 16

NC = 2
NS = 16
NW = NC * NS
CC = 64
NCHT = EE // CC
CB, CX = NCHT // NW, NCHT % NW
NP = 10240
ND = NP // HH
SROWS = NP // NS
DROWS = ND // NS
ZR = 128


def _edge_body(xl_hbm, xr_hbm, src_hbm, dst_hbm, att_hbm, za_hbm,
               onum_hbm, oden_hbm,
               acc, accd, srcv, dstv, dst2v, xlv, xrv, wrowv, wrow2v, wv,
               attv, sem1, sem2):
    ci = lax.axis_index("c")
    si = lax.axis_index("s")
    wid = si * NC + ci

    pltpu.sync_copy(att_hbm, attv)

    for k in range(SROWS // ZR):
        pltpu.sync_copy(za_hbm, acc.at[pl.ds(si * SROWS + k * ZR, ZR)])
    pltpu.sync_copy(za_hbm.at[pl.ds(0, DROWS)], accd.at[pl.ds(si * DROWS, DROWS)])
    plsc.subcore_barrier()

    nch = CB + jnp.where(wid < CX, 1, 0)
    start = wid * CB + jnp.minimum(wid, CX)

    def chunk_body(j, carry):
        base = (start + j) * CC
        pltpu.sync_copy(src_hbm.at[pl.ds(base, CC)], srcv)
        pltpu.sync_copy(dst_hbm.at[pl.ds(base, CC)], dstv)
        cp1 = pltpu.async_copy(xl_hbm.at[srcv], xlv, sem1)
        cp2 = pltpu.async_copy(xr_hbm.at[dstv], xrv, sem2)

        def d2_body(k, c3):
            dst2v[pl.ds(k * 16, 16)] = lax.shift_right_logical(
                dstv[pl.ds(k * 16, 16)], 3)
            return c3

        lax.fori_loop(0, CC // 16, d2_body, 0)
        cp1.wait()
        cp2.wait()

        def edge_body(i, c2):
            lane = lax.iota(jnp.int32, 16)
            mask8f = jnp.where(lane < HH, 1.0, 0.0)
            xlh = [xlv[i, pl.ds(h * DHD, DHD)] for h in range(HH)]
            svec = jnp.zeros((16,), jnp.float32)
            for h in range(HH):
                t = xlh[h] + xrv[i, pl.ds(h * DHD, DHD)]
                t = jnp.maximum(t, t * 0.2) * attv[h, :]
                sc = lax.broadcast(jnp.sum(t), (16,))
                svec = jnp.where(lane == h, sc, svec)
            wvec = jnp.exp(svec)
            wv[pl.ds(i * 16, 16)] = wvec
            dvec = wvec * mask8f
            ib = i * 16
            for h in range(HH):
                wb = plsc.load_gather(wv, [lax.broadcast(ib + h, (16,))])
                wrowv[i, pl.ds(h * DHD, DHD)] = xlh[h] * wb
            dstb = plsc.load_gather(dstv, [lax.broadcast(i, (16,))])
            g = dstb & 7
            zf = jnp.zeros((16,), jnp.float32)
            for w in range(HH):
                wrow2v[i, pl.ds(w * DHD, DHD)] = jnp.where(g == w, dvec, zf)
            return c2

        lax.fori_loop(0, CC, edge_body, 0)
        pltpu.sync_copy(wrowv, acc.at[dstv], add=True)
        pltpu.sync_copy(wrow2v, accd.at[dst2v], add=True)
        return carry

    lax.fori_loop(0, nch, chunk_body, 0)
    plsc.subcore_barrier()
    pltpu.sync_copy(acc.at[pl.ds(si * SROWS, SROWS)],
                    onum_hbm.at[ci, pl.ds(si * SROWS, SROWS)])
    pltpu.sync_copy(accd.at[pl.ds(si * DROWS, DROWS)],
                    oden_hbm.at[ci, pl.ds(si * DROWS, DROWS)])


def _edge_partials(xl, xr, src, dst, att, za):
    mesh = plsc.VectorSubcoreMesh(core_axis_name="c", subcore_axis_name="s")
    f = functools.partial(
        pl.kernel,
        mesh=mesh,
        compiler_params=pltpu.CompilerParams(needs_layout_passes=False),
        out_type=[
            jax.ShapeDtypeStruct((NC, NP, DD), jnp.float32),
            jax.ShapeDtypeStruct((NC, ND, DD), jnp.float32),
        ],
        scratch_types=[
            pltpu.VMEM_SHARED((NP, DD), jnp.float32),
            pltpu.VMEM_SHARED((ND, DD), jnp.float32),
            pltpu.VMEM((CC,), jnp.int32),
            pltpu.VMEM((CC,), jnp.int32),
            pltpu.VMEM((CC,), jnp.int32),
            pltpu.VMEM((CC, DD), jnp.float32),
            pltpu.VMEM((CC, DD), jnp.float32),
            pltpu.VMEM((CC, DD), jnp.float32),
            pltpu.VMEM((CC, DD), jnp.float32),
            pltpu.VMEM((CC * 16,), jnp.float32),
            pltpu.VMEM((HH, DHD), jnp.float32),
            pltpu.SemaphoreType.DMA,
            pltpu.SemaphoreType.DMA,
        ],
    )(_edge_body)
    return f(xl, xr, src, dst, att, za)


def _pre_body(x_ref, ln1_ref, wl_ref, bl_ref, wr_ref, br_ref, xl_ref, xr_ref):
    xb = x_ref[...]
    mu = jnp.mean(xb, axis=1, keepdims=True)
    xc = xb - mu
    var = jnp.mean(xc * xc, axis=1, keepdims=True)
    hb = xc * lax.rsqrt(var + 1e-5) * ln1_ref[...]
    xl_ref[...] = jnp.dot(hb, wl_ref[...], preferred_element_type=jnp.float32) + bl_ref[...]
    xr_ref[...] = jnp.dot(hb, wr_ref[...], preferred_element_type=jnp.float32) + br_ref[...]


def _post_body(x_ref, pn_ref, pd_ref, bias_ref, ln2_ref, wfc_ref, wproj_ref,
               sel_ref, o_ref):
    num = pn_ref[0] + pn_ref[1]
    den = pd_ref[0] + pd_ref[1]
    inv = 1.0 / (den + 1e-16)
    expand = jnp.dot(inv, sel_ref[...], preferred_element_type=jnp.float32)
    x1 = x_ref[...] + num * expand + bias_ref[...]
    mu = jnp.mean(x1, axis=1, keepdims=True)
    xc = x1 - mu
    var = jnp.mean(xc * xc, axis=1, keepdims=True)
    h2 = xc * lax.rsqrt(var + 1e-5) * ln2_ref[...]
    g = jnp.dot(h2, wfc_ref[...], preferred_element_type=jnp.float32)
    g = g * 0.5 * (1.0 + lax.erf(g * 0.7071067811865476))
    o_ref[...] = x1 + jnp.dot(g, wproj_ref[...], preferred_element_type=jnp.float32)


_BR = 400


def _pre(x, ln1_w, Wl, bl, Wr, br):
    grid = NN // _BR
    full = lambda s: pl.BlockSpec(s, lambda i: (0,) * len(s))
    return pl.pallas_call(
        _pre_body,
        grid=(grid,),
        in_specs=[
            pl.BlockSpec((_BR, DD), lambda i: (i, 0)),
            full((1, DD)), full((DD, DD)), full((1, DD)),
            full((DD, DD)), full((1, DD)),
        ],
        out_specs=[pl.BlockSpec((_BR, DD), lambda i: (i, 0))] * 2,
        out_shape=[jax.ShapeDtypeStruct((NN, DD), jnp.float32)] * 2,
    )(x, ln1_w.reshape(1, DD), Wl, bl.reshape(1, DD), Wr, br.reshape(1, DD))


def _post(x, pnum, pden, attn_bias, ln2_w, W_fc, W_proj, sel):
    grid = NN // _BR
    full = lambda s: pl.BlockSpec(s, lambda i: (0,) * len(s))
    return pl.pallas_call(
        _post_body,
        grid=(grid,),
        in_specs=[
            pl.BlockSpec((_BR, DD), lambda i: (i, 0)),
            pl.BlockSpec((NC, _BR, DD), lambda i: (0, i, 0)),
            pl.BlockSpec((NC, _BR, HH), lambda i: (0, i, 0)),
            full((1, DD)), full((1, DD)),
            full((DD, 4 * DD)), full((4 * DD, DD)), full((HH, DD)),
        ],
        out_specs=pl.BlockSpec((_BR, DD), lambda i: (i, 0)),
        out_shape=jax.ShapeDtypeStruct((NN, DD), jnp.float32),
    )(x, pnum, pden, attn_bias.reshape(1, DD), ln2_w.reshape(1, DD),
      W_fc, W_proj, sel)


def kernel(x, edges_idx, ln1_w, Wl, bl, Wr, br, att, attn_bias, ln2_w, W_fc, W_proj):
    src = edges_idx[0]
    dst = edges_idx[1]
    xl, xr = _pre(x, ln1_w, Wl, bl, Wr, br)
    za = jnp.zeros((ZR, DD), jnp.float32)
    pnum, pden = _edge_partials(xl, xr, src, dst, att, za)
    sel = (jnp.arange(DD)[None, :] // DHD == jnp.arange(HH)[:, None]
           ).astype(jnp.float32)
    pden = pden.reshape(NC, NP, DHD)[:, :NN, :HH]
    return _post(x, pnum[:, :NN], pden, attn_bias, ln2_w, W_fc, W_proj, sel)


# speedup vs baseline: 43.0632x; 1.3495x over previous
import functools

import jax
import jax.numpy as jnp
from jax import lax
from jax.experimental import pallas as pl
from jax.experimental.pallas import tpu as pltpu
from jax.experimental.pallas import tpu_sc as plsc

NN = 10000
EE = 320000
DD = 128
HH = 8
DHD = 16

NC = 2
NS = 16
NW = NC * NS
CC = 64
NCHT = EE // CC
CB, CX = NCHT // NW, NCHT % NW
NP = 10240
ND = NP // HH
SROWS = NP // NS
DROWS = ND // NS
ZR = 128


def _edge_body(xl_hbm, xr_hbm, src_hbm, dst_hbm, att_hbm, za_hbm,
               onum_hbm, oden_hbm,
               acc, accd, srcv, dstv, dst2v, xlv, xrv, wrowv, wrow2v, wv,
               attv, sem1, sem2):
    ci = lax.axis_index("c")
    si = lax.axis_index("s")
    wid = si * NC + ci

    pltpu.sync_copy(att_hbm, attv)
    att_r = [attv[h, :] for h in range(HH)]
    lane = lax.iota(jnp.int32, 16)
    mask8f = jnp.where(lane < HH, 1.0, 0.0)
    zf = jnp.zeros((16,), jnp.float32)

    for k in range(SROWS // ZR):
        pltpu.sync_copy(za_hbm, acc.at[pl.ds(si * SROWS + k * ZR, ZR)])
    pltpu.sync_copy(za_hbm.at[pl.ds(0, DROWS)], accd.at[pl.ds(si * DROWS, DROWS)])
    plsc.subcore_barrier()

    nch = CB + jnp.where(wid < CX, 1, 0)
    start = wid * CB + jnp.minimum(wid, CX)

    def chunk_body(j, carry):
        base = (start + j) * CC
        pltpu.sync_copy(src_hbm.at[pl.ds(base, CC)], srcv)
        pltpu.sync_copy(dst_hbm.at[pl.ds(base, CC)], dstv)
        cp1 = pltpu.async_copy(xl_hbm.at[srcv], xlv, sem1)
        cp2 = pltpu.async_copy(xr_hbm.at[dstv], xrv, sem2)

        def d2_body(k, c3):
            dst2v[pl.ds(k * 16, 16)] = lax.shift_right_logical(
                dstv[pl.ds(k * 16, 16)], 3)
            return c3

        lax.fori_loop(0, CC // 16, d2_body, 0)
        cp1.wait()
        cp2.wait()

        @plsc.parallel_loop(0, CC, 1, unroll=4)
        def edge_body(i):
            xlh = [xlv[i, pl.ds(h * DHD, DHD)] for h in range(HH)]
            svec = jnp.zeros((16,), jnp.float32)
            for h in range(HH):
                t = xlh[h] + xrv[i, pl.ds(h * DHD, DHD)]
                t = jnp.maximum(t, t * 0.2) * att_r[h]
                sc = lax.broadcast(jnp.sum(t), (16,))
                svec = jnp.where(lane == h, sc, svec)
            wvec = jnp.exp(svec)
            wv[pl.ds(i * 16, 16)] = wvec
            dvec = wvec * mask8f
            ib = i * 16
            for h in range(HH):
                wb = plsc.load_gather(wv, [lax.broadcast(ib + h, (16,))])
                wrowv[i, pl.ds(h * DHD, DHD)] = xlh[h] * wb
            dstb = plsc.load_gather(dstv, [lax.broadcast(i, (16,))])
            g = dstb & 7
            for w in range(HH):
                wrow2v[i, pl.ds(w * DHD, DHD)] = jnp.where(g == w, dvec, zf)
        pltpu.sync_copy(wrowv, acc.at[dstv], add=True)
        pltpu.sync_copy(wrow2v, accd.at[dst2v], add=True)
        return carry

    lax.fori_loop(0, nch, chunk_body, 0)
    plsc.subcore_barrier()
    pltpu.sync_copy(acc.at[pl.ds(si * SROWS, SROWS)],
                    onum_hbm.at[ci, pl.ds(si * SROWS, SROWS)])
    pltpu.sync_copy(accd.at[pl.ds(si * DROWS, DROWS)],
                    oden_hbm.at[ci, pl.ds(si * DROWS, DROWS)])


def _edge_partials(xl, xr, src, dst, att, za):
    mesh = plsc.VectorSubcoreMesh(core_axis_name="c", subcore_axis_name="s")
    f = functools.partial(
        pl.kernel,
        mesh=mesh,
        compiler_params=pltpu.CompilerParams(needs_layout_passes=False),
        out_type=[
            jax.ShapeDtypeStruct((NC, NP, DD), jnp.float32),
            jax.ShapeDtypeStruct((NC, ND, DD), jnp.float32),
        ],
        scratch_types=[
            pltpu.VMEM_SHARED((NP, DD), jnp.float32),
            pltpu.VMEM_SHARED((ND, DD), jnp.float32),
            pltpu.VMEM((CC,), jnp.int32),
            pltpu.VMEM((CC,), jnp.int32),
            pltpu.VMEM((CC,), jnp.int32),
            pltpu.VMEM((CC, DD), jnp.float32),
            pltpu.VMEM((CC, DD), jnp.float32),
            pltpu.VMEM((CC, DD), jnp.float32),
            pltpu.VMEM((CC, DD), jnp.float32),
            pltpu.VMEM((CC * 16,), jnp.float32),
            pltpu.VMEM((HH, DHD), jnp.float32),
            pltpu.SemaphoreType.DMA,
            pltpu.SemaphoreType.DMA,
        ],
    )(_edge_body)
    return f(xl, xr, src, dst, att, za)


def _pre_body(x_ref, ln1_ref, wl_ref, bl_ref, wr_ref, br_ref, xl_ref, xr_ref):
    xb = x_ref[...]
    mu = jnp.mean(xb, axis=1, keepdims=True)
    xc = xb - mu
    var = jnp.mean(xc * xc, axis=1, keepdims=True)
    hb = xc * lax.rsqrt(var + 1e-5) * ln1_ref[...]
    xl_ref[...] = jnp.dot(hb, wl_ref[...], preferred_element_type=jnp.float32) + bl_ref[...]
    xr_ref[...] = jnp.dot(hb, wr_ref[...], preferred_element_type=jnp.float32) + br_ref[...]


def _post_body(x_ref, pn_ref, pd_ref, bias_ref, ln2_ref, wfc_ref, wproj_ref,
               sel_ref, o_ref):
    num = pn_ref[0] + pn_ref[1]
    den = pd_ref[0] + pd_ref[1]
    inv = 1.0 / (den + 1e-16)
    expand = jnp.dot(inv, sel_ref[...], preferred_element_type=jnp.float32)
    x1 = x_ref[...] + num * expand + bias_ref[...]
    mu = jnp.mean(x1, axis=1, keepdims=True)
    xc = x1 - mu
    var = jnp.mean(xc * xc, axis=1, keepdims=True)
    h2 = xc * lax.rsqrt(var + 1e-5) * ln2_ref[...]
    g = jnp.dot(h2, wfc_ref[...], preferred_element_type=jnp.float32)
    g = g * 0.5 * (1.0 + lax.erf(g * 0.7071067811865476))
    o_ref[...] = x1 + jnp.dot(g, wproj_ref[...], preferred_element_type=jnp.float32)


_BR = 400


def _pre(x, ln1_w, Wl, bl, Wr, br):
    grid = NN // _BR
    full = lambda s: pl.BlockSpec(s, lambda i: (0,) * len(s))
    return pl.pallas_call(
        _pre_body,
        grid=(grid,),
        in_specs=[
            pl.BlockSpec((_BR, DD), lambda i: (i, 0)),
            full((1, DD)), full((DD, DD)), full((1, DD)),
            full((DD, DD)), full((1, DD)),
        ],
        out_specs=[pl.BlockSpec((_BR, DD), lambda i: (i, 0))] * 2,
        out_shape=[jax.ShapeDtypeStruct((NN, DD), jnp.float32)] * 2,
    )(x, ln1_w.reshape(1, DD), Wl, bl.reshape(1, DD), Wr, br.reshape(1, DD))


def _post(x, pnum, pden, attn_bias, ln2_w, W_fc, W_proj, sel):
    grid = NN // _BR
    full = lambda s: pl.BlockSpec(s, lambda i: (0,) * len(s))
    return pl.pallas_call(
        _post_body,
        grid=(grid,),
        in_specs=[
            pl.BlockSpec((_BR, DD), lambda i: (i, 0)),
            pl.BlockSpec((NC, _BR, DD), lambda i: (0, i, 0)),
            pl.BlockSpec((NC, _BR, HH), lambda i: (0, i, 0)),
            full((1, DD)), full((1, DD)),
            full((DD, 4 * DD)), full((4 * DD, DD)), full((HH, DD)),
        ],
        out_specs=pl.BlockSpec((_BR, DD), lambda i: (i, 0)),
        out_shape=jax.ShapeDtypeStruct((NN, DD), jnp.float32),
    )(x, pnum, pden, attn_bias.reshape(1, DD), ln2_w.reshape(1, DD),
      W_fc, W_proj, sel)


def kernel(x, edges_idx, ln1_w, Wl, bl, Wr, br, att, attn_bias, ln2_w, W_fc, W_proj):
    src = edges_idx[0]
    dst = edges_idx[1]
    xl, xr = _pre(x, ln1_w, Wl, bl, Wr, br)
    za = jnp.zeros((ZR, DD), jnp.float32)
    pnum, pden = _edge_partials(xl, xr, src, dst, att, za)
    sel = (jnp.arange(DD)[None, :] // DHD == jnp.arange(HH)[:, None]
           ).astype(jnp.float32)
    pden = pden.reshape(NC, NP, DHD)[:, :NN, :HH]
    return _post(x, pnum[:, :NN], pden, attn_bias, ln2_w, W_fc, W_proj, sel)


# speedup vs baseline: 50.9872x; 1.1840x over previous
import functools

import jax
import jax.numpy as jnp
from jax import lax
from jax.experimental import pallas as pl
from jax.experimental.pallas import tpu as pltpu
from jax.experimental.pallas import tpu_sc as plsc

NN = 10000
EE = 320000
DD = 128
HH = 8
DHD = 16

NC = 2
NS = 16
NW = NC * NS
CC = 32
NCHT = EE // CC
CB, CX = NCHT // NW, NCHT % NW
PAIRS = (CB + 2) // 2
NP = 10240
ND = NP // HH
SROWS = NP // NS
DROWS = ND // NS
ZR = 128


def _edge_body(xl_hbm, xr_hbm, src_hbm, dst_hbm, att_hbm, za_hbm,
               onum_hbm, oden_hbm,
               acc, accd,
               srcv0, srcv1, dstv0, dstv1, d2v0, d2v1,
               sd0, sd1, sd20, sd21,
               xlv0, xlv1, xrv0, xrv1, wr0, wr1, w20, w21,
               wv, attv, semg0, semg1, sems0, sems1):
    ci = lax.axis_index("c")
    si = lax.axis_index("s")
    wid = si * NC + ci

    pltpu.sync_copy(att_hbm, attv)
    att_r = [attv[h, :] for h in range(HH)]
    lane = lax.iota(jnp.int32, 16)
    mask8f = jnp.where(lane < HH, 1.0, 0.0)
    zf = jnp.zeros((16,), jnp.float32)

    for k in range(SROWS // ZR):
        pltpu.sync_copy(za_hbm, acc.at[pl.ds(si * SROWS + k * ZR, ZR)])
    pltpu.sync_copy(za_hbm.at[pl.ds(0, DROWS)], accd.at[pl.ds(si * DROWS, DROWS)])
    plsc.subcore_barrier()

    nch = CB + jnp.where(wid < CX, 1, 0)
    start = wid * CB + jnp.minimum(wid, CX)

    def cbase(c):
        return jnp.where(c < nch, (start + c) * CC, 0)

    def load_idx(c, srcv, dstv, d2v):
        b = cbase(c)
        pltpu.sync_copy(src_hbm.at[pl.ds(b, CC)], srcv)
        pltpu.sync_copy(dst_hbm.at[pl.ds(b, CC)], dstv)

        @plsc.parallel_loop(0, CC // 16, 1)
        def d2_body(k):
            d2v[pl.ds(k * 16, 16)] = lax.shift_right_logical(
                dstv[pl.ds(k * 16, 16)], 3)

    def start_gather(srcv, dstv, xlv, xrv, semg):
        pltpu.async_copy(xl_hbm.at[srcv], xlv, semg)
        pltpu.async_copy(xr_hbm.at[dstv], xrv, semg)

    def wait_gather(srcv, dstv, xlv, xrv, semg):
        pltpu.make_async_copy(xl_hbm.at[srcv], xlv, semg).wait()
        pltpu.make_async_copy(xr_hbm.at[dstv], xrv, semg).wait()

    def snap_idx(dstv, d2v, sd, sd2):
        @plsc.parallel_loop(0, CC // 16, 1)
        def cp_body(k):
            sd[pl.ds(k * 16, 16)] = dstv[pl.ds(k * 16, 16)]
            sd2[pl.ds(k * 16, 16)] = d2v[pl.ds(k * 16, 16)]

    def start_scatter(wrowv, wrow2v, sd, sd2, sems):
        pltpu.async_copy(wrowv, acc.at[sd], sems, add=True)
        pltpu.async_copy(wrow2v, accd.at[sd2], sems, add=True)

    def wait_scatter(wrowv, wrow2v, sd, sd2, sems):
        pltpu.make_async_copy(wrowv, acc.at[sd], sems).wait()
        pltpu.make_async_copy(wrow2v, accd.at[sd2], sems).wait()

    def compute(c, xlv, xrv, dstv, wrowv, wrow2v):
        af = jnp.where(lax.broadcast(c, (16,)) < lax.broadcast(nch, (16,)),
                       1.0, 0.0)

        @plsc.parallel_loop(0, CC, 1, unroll=2)
        def edge_body(i):
            xlh = [xlv[i, pl.ds(h * DHD, DHD)] for h in range(HH)]
            svec = jnp.zeros((16,), jnp.float32)
            for h in range(HH):
                t = xlh[h] + xrv[i, pl.ds(h * DHD, DHD)]
                t = jnp.maximum(t, t * 0.2) * att_r[h]
                sc = lax.broadcast(jnp.sum(t), (16,))
                svec = jnp.where(lane == h, sc, svec)
            wvec = jnp.exp(svec) * af
            wv[pl.ds(i * 16, 16)] = wvec
            dvec = wvec * mask8f
            ib = i * 16
            for h in range(HH):
                wb = plsc.load_gather(wv, [lax.broadcast(ib + h, (16,))])
                wrowv[i, pl.ds(h * DHD, DHD)] = xlh[h] * wb
            dstb = plsc.load_gather(dstv, [lax.broadcast(i, (16,))])
            g = dstb & 7
            for w in range(HH):
                wrow2v[i, pl.ds(w * DHD, DHD)] = jnp.where(g == w, dvec, zf)

    load_idx(0, srcv0, dstv0, d2v0)
    start_gather(srcv0, dstv0, xlv0, xrv0, semg0)
    for wb_ in (wr0, wr1, w20, w21):
        pltpu.sync_copy(za_hbm.at[pl.ds(0, CC)], wb_)
    snap_idx(dstv0, d2v0, sd0, sd20)
    snap_idx(dstv0, d2v0, sd1, sd21)
    start_scatter(wr0, w20, sd0, sd20, sems0)
    start_scatter(wr1, w21, sd1, sd21, sems1)

    def pair_body(g, carry):
        cA = 2 * g
        cB = cA + 1
        wait_gather(srcv0, dstv0, xlv0, xrv0, semg0)
        load_idx(cB, srcv1, dstv1, d2v1)
        start_gather(srcv1, dstv1, xlv1, xrv1, semg1)
        wait_scatter(wr0, w20, sd0, sd20, sems0)
        compute(cA, xlv0, xrv0, dstv0, wr0, w20)
        snap_idx(dstv0, d2v0, sd0, sd20)
        start_scatter(wr0, w20, sd0, sd20, sems0)
        wait_gather(srcv1, dstv1, xlv1, xrv1, semg1)
        load_idx(cA + 2, srcv0, dstv0, d2v0)
        start_gather(srcv0, dstv0, xlv0, xrv0, semg0)
        wait_scatter(wr1, w21, sd1, sd21, sems1)
        compute(cB, xlv1, xrv1, dstv1, wr1, w21)
        snap_idx(dstv1, d2v1, sd1, sd21)
        start_scatter(wr1, w21, sd1, sd21, sems1)
        return carry

    lax.fori_loop(0, PAIRS, pair_body, 0)
    wait_gather(srcv0, dstv0, xlv0, xrv0, semg0)
    wait_scatter(wr0, w20, sd0, sd20, sems0)
    wait_scatter(wr1, w21, sd1, sd21, sems1)
    plsc.subcore_barrier()
    pltpu.sync_copy(acc.at[pl.ds(si * SROWS, SROWS)],
                    onum_hbm.at[ci, pl.ds(si * SROWS, SROWS)])
    pltpu.sync_copy(accd.at[pl.ds(si * DROWS, DROWS)],
                    oden_hbm.at[ci, pl.ds(si * DROWS, DROWS)])


def _edge_partials(xl, xr, src, dst, att, za):
    mesh = plsc.VectorSubcoreMesh(core_axis_name="c", subcore_axis_name="s")
    f = functools.partial(
        pl.kernel,
        mesh=mesh,
        compiler_params=pltpu.CompilerParams(needs_layout_passes=False),
        out_type=[
            jax.ShapeDtypeStruct((NC, NP, DD), jnp.float32),
            jax.ShapeDtypeStruct((NC, ND, DD), jnp.float32),
        ],
        scratch_types=[
            pltpu.VMEM_SHARED((NP, DD), jnp.float32),
            pltpu.VMEM_SHARED((ND, DD), jnp.float32),
            pltpu.VMEM((CC,), jnp.int32),
            pltpu.VMEM((CC,), jnp.int32),
            pltpu.VMEM((CC,), jnp.int32),
            pltpu.VMEM((CC,), jnp.int32),
            pltpu.VMEM((CC,), jnp.int32),
            pltpu.VMEM((CC,), jnp.int32),
            pltpu.VMEM((CC,), jnp.int32),
            pltpu.VMEM((CC,), jnp.int32),
            pltpu.VMEM((CC,), jnp.int32),
            pltpu.VMEM((CC,), jnp.int32),
            pltpu.VMEM((CC, DD), jnp.float32),
            pltpu.VMEM((CC, DD), jnp.float32),
            pltpu.VMEM((CC, DD), jnp.float32),
            pltpu.VMEM((CC, DD), jnp.float32),
            pltpu.VMEM((CC, DD), jnp.float32),
            pltpu.VMEM((CC, DD), jnp.float32),
            pltpu.VMEM((CC, DD), jnp.float32),
            pltpu.VMEM((CC, DD), jnp.float32),
            pltpu.VMEM((CC * 16,), jnp.float32),
            pltpu.VMEM((HH, DHD), jnp.float32),
            pltpu.SemaphoreType.DMA,
            pltpu.SemaphoreType.DMA,
            pltpu.SemaphoreType.DMA,
            pltpu.SemaphoreType.DMA,
        ],
    )(_edge_body)
    return f(xl, xr, src, dst, att, za)


def _pre_body(x_ref, ln1_ref, wl_ref, bl_ref, wr_ref, br_ref, xl_ref, xr_ref):
    xb = x_ref[...]
    mu = jnp.mean(xb, axis=1, keepdims=True)
    xc = xb - mu
    var = jnp.mean(xc * xc, axis=1, keepdims=True)
    hb = xc * lax.rsqrt(var + 1e-5) * ln1_ref[...]
    xl_ref[...] = jnp.dot(hb, wl_ref[...], preferred_element_type=jnp.float32) + bl_ref[...]
    xr_ref[...] = jnp.dot(hb, wr_ref[...], preferred_element_type=jnp.float32) + br_ref[...]


def _post_body(x_ref, pn_ref, pd_ref, bias_ref, ln2_ref, wfc_ref, wproj_ref,
               sel_ref, o_ref):
    num = pn_ref[0] + pn_ref[1]
    den = pd_ref[0] + pd_ref[1]
    inv = 1.0 / (den + 1e-16)
    expand = jnp.dot(inv, sel_ref[...], preferred_element_type=jnp.float32)
    x1 = x_ref[...] + num * expand + bias_ref[...]
    mu = jnp.mean(x1, axis=1, keepdims=True)
    xc = x1 - mu
    var = jnp.mean(xc * xc, axis=1, keepdims=True)
    h2 = xc * lax.rsqrt(var + 1e-5) * ln2_ref[...]
    g = jnp.dot(h2, wfc_ref[...], preferred_element_type=jnp.float32)
    g = g * 0.5 * (1.0 + lax.erf(g * 0.7071067811865476))
    o_ref[...] = x1 + jnp.dot(g, wproj_ref[...], preferred_element_type=jnp.float32)


_BR = 400


def _pre(x, ln1_w, Wl, bl, Wr, br):
    grid = NN // _BR
    full = lambda s: pl.BlockSpec(s, lambda i: (0,) * len(s))
    return pl.pallas_call(
        _pre_body,
        grid=(grid,),
        in_specs=[
            pl.BlockSpec((_BR, DD), lambda i: (i, 0)),
            full((1, DD)), full((DD, DD)), full((1, DD)),
            full((DD, DD)), full((1, DD)),
        ],
        out_specs=[pl.BlockSpec((_BR, DD), lambda i: (i, 0))] * 2,
        out_shape=[jax.ShapeDtypeStruct((NN, DD), jnp.float32)] * 2,
    )(x, ln1_w.reshape(1, DD), Wl, bl.reshape(1, DD), Wr, br.reshape(1, DD))


def _post(x, pnum, pden, attn_bias, ln2_w, W_fc, W_proj, sel):
    grid = NN // _BR
    full = lambda s: pl.BlockSpec(s, lambda i: (0,) * len(s))
    return pl.pallas_call(
        _post_body,
        grid=(grid,),
        in_specs=[
            pl.BlockSpec((_BR, DD), lambda i: (i, 0)),
            pl.BlockSpec((NC, _BR, DD), lambda i: (0, i, 0)),
            pl.BlockSpec((NC, _BR, HH), lambda i: (0, i, 0)),
            full((1, DD)), full((1, DD)),
            full((DD, 4 * DD)), full((4 * DD, DD)), full((HH, DD)),
        ],
        out_specs=pl.BlockSpec((_BR, DD), lambda i: (i, 0)),
        out_shape=jax.ShapeDtypeStruct((NN, DD), jnp.float32),
    )(x, pnum, pden, attn_bias.reshape(1, DD), ln2_w.reshape(1, DD),
      W_fc, W_proj, sel)


def kernel(x, edges_idx, ln1_w, Wl, bl, Wr, br, att, attn_bias, ln2_w, W_fc, W_proj):
    src = edges_idx[0]
    dst = edges_idx[1]
    xl, xr = _pre(x, ln1_w, Wl, bl, Wr, br)
    za = jnp.zeros((ZR, DD), jnp.float32)
    pnum, pden = _edge_partials(xl, xr, src, dst, att, za)
    sel = (jnp.arange(DD)[None, :] // DHD == jnp.arange(HH)[:, None]
           ).astype(jnp.float32)
    pden = pden.reshape(NC, NP, DHD)[:, :NN, :HH]
    return _post(x, pnum[:, :NN], pden, attn_bias, ln2_w, W_fc, W_proj, sel)
